# Initial kernel scaffold; baseline (speedup 1.0000x reference)
import jax
import jax.numpy as jnp
from jax.experimental import pallas as pl


def kernel(x, edge_index, edge_attr, node_weight, W1, Wd, W2, Wn1, Wnd, Wn2, Wo1, bo1, Wo2, bo2):
    raise NotImplementedError("write your pallas kernel here")


# speedup vs baseline: 37.1524x; 37.1524x over previous
import functools

import jax
import jax.numpy as jnp
from jax import lax
from jax.experimental import pallas as pl
from jax.experimental.pallas import tpu as pltpu
from jax.experimental.pallas import tpu_sc as plsc

NC = 2
NS = 16
GW = 64
IW = 125


def _make_gather(n_nodes, n_edges):
    ew = n_edges // (NC * NS)
    ca = 1000
    sub = ca // IW
    mesh = plsc.VectorSubcoreMesh(core_axis_name="c", subcore_axis_name="s")

    @functools.partial(
        pl.kernel, mesh=mesh,
        out_type=jax.ShapeDtypeStruct((n_edges, GW), jnp.float32),
        scratch_types=[
            pltpu.VMEM((sub, IW), jnp.int32),
            pltpu.VMEM((ca, GW), jnp.float32),
            pltpu.SemaphoreType.DMA,
        ],
        compiler_params=pltpu.CompilerParams(use_tc_tiling_on_sc=False),
    )
    def gather_k(xt_hbm, src2d_hbm, out_hbm, idx_v, rows_v, sem):
        wid = lax.axis_index("s") * NC + lax.axis_index("c")
        base = wid * ew

        def chunk(i, carry):
            off = pl.multiple_of(base + i * ca, 8)
            pltpu.sync_copy(src2d_hbm.at[pl.ds(pl.multiple_of(off // IW, 8), sub)],
                            idx_v)
            cps = [
                pltpu.async_copy(
                    xt_hbm.at[idx_v.at[j]],
                    rows_v.at[pl.ds(j * IW, IW)], sem)
                for j in range(sub)
            ]
            for cp in cps:
                cp.wait()
            pltpu.sync_copy(rows_v, out_hbm.at[pl.ds(off, ca)])
            return carry

        lax.fori_loop(0, ew // ca, chunk, 0)

    return gather_k


def _make_scatter(n_nodes, n_edges):
    es = n_edges // NS
    cc = 500
    sub = cc // IW
    nra = (-(-n_nodes // NS) + 7) // 8 * 8
    nrl = n_nodes - (NS - 1) * nra
    mesh = plsc.VectorSubcoreMesh(core_axis_name="c", subcore_axis_name="s")

    @functools.partial(
        pl.kernel, mesh=mesh,
        out_type=jax.ShapeDtypeStruct((NC, n_nodes, 32), jnp.float32),
        scratch_types=[
            pltpu.VMEM_SHARED((n_nodes, 32), jnp.float32),
            pltpu.VMEM((sub, IW), jnp.int32),
            pltpu.VMEM((cc, 32), jnp.float32),
            pltpu.SemaphoreType.DMA,
        ],
        compiler_params=pltpu.CompilerParams(use_tc_tiling_on_sc=False),
    )
    def scatter_k(msgs_hbm, dst2d_hbm, zeros_hbm, out_hbm, acc, idx_v, buf, sem):
        c = lax.axis_index("c")
        s = lax.axis_index("s")

        @pl.when(s < NS - 1)
        def _():
            pltpu.sync_copy(zeros_hbm,
                            acc.at[pl.ds(pl.multiple_of(s * nra, 8), nra)])

        @pl.when(s == NS - 1)
        def _():
            pltpu.sync_copy(zeros_hbm.at[pl.ds(0, nrl)],
                            acc.at[pl.ds((NS - 1) * nra, nrl)])

        plsc.subcore_barrier()

        base = s * es

        def chunk(i, carry):
            off = pl.multiple_of(base + i * cc, 8)
            pltpu.sync_copy(dst2d_hbm.at[pl.ds(pl.multiple_of(off // IW, 8), sub)],
                            idx_v)
            pltpu.sync_copy(msgs_hbm.at[c].at[pl.ds(off, cc)], buf)
            cps = [
                pltpu.async_copy(
                    buf.at[pl.ds(j * IW, IW)],
                    acc.at[idx_v.at[j]], sem, add=True)
                for j in range(sub)
            ]
            for cp in cps:
                cp.wait()
            return carry

        lax.fori_loop(0, es // cc, chunk, 0)
        plsc.subcore_barrier()

        @pl.when(s < NS - 1)
        def _():
            off = pl.multiple_of(s * nra, 8)
            pltpu.sync_copy(acc.at[pl.ds(off, nra)],
                            out_hbm.at[c].at[pl.ds(off, nra)])

        @pl.when(s == NS - 1)
        def _():
            pltpu.sync_copy(acc.at[pl.ds((NS - 1) * nra, nrl)],
                            out_hbm.at[c].at[pl.ds((NS - 1) * nra, nrl)])

    return scatter_k


def _edge_body(g_ref, ea_ref, k1_ref, kd_ref, k2_ref, ss_ref, st_ref, o_ref):
    g = g_ref[...]
    w = g[:, 48:49]
    cat = jnp.concatenate([g[:, :48], ea_ref[...]], axis=1)
    m1 = jnp.dot(cat, k1_ref[...], preferred_element_type=jnp.float32)
    d = jnp.dot(m1, kd_ref[...], preferred_element_type=jnp.float32)
    t = jnp.dot(jnp.concatenate([d * d, m1 * d], axis=1), ss_ref[...],
                preferred_element_type=jnp.float32)
    dn2 = t[:, :16]
    t16 = t[:, 16:32]
    q = jnp.where(t16 >= 0.0, 0.0, t16 / dn2)
    r = m1 - d * jnp.dot(q, st_ref[...], preferred_element_type=jnp.float32)
    m = jnp.dot(r, k2_ref[...], preferred_element_type=jnp.float32)
    wm = m * w
    o_ref[0, ...] = wm[:, :32]
    o_ref[1, ...] = jnp.concatenate(
        [wm[:, 32:48], w, jnp.zeros((wm.shape[0], 15), jnp.float32)], axis=1)


def _node_body(x_ref, ms_ref, kn1_ref, knd_ref, kn2_ref, ss_ref, st_ref,
               wo1_ref, bo1_ref, wo2_ref, bo2_ref, o_ref):
    m0 = ms_ref[0, ...]
    m1h = ms_ref[1, ...]
    den = m1h[:, 16:17]
    num = jnp.concatenate([m0, m1h[:, :16]], axis=1)
    agg = num / (den + 1e-12)
    cat = jnp.concatenate([x_ref[...], agg], axis=1)
    h1 = jnp.dot(cat, kn1_ref[...], preferred_element_type=jnp.float32)
    d = jnp.dot(h1, knd_ref[...], preferred_element_type=jnp.float32)
    t = jnp.dot(jnp.concatenate([d * d, h1 * d], axis=1), ss_ref[...],
                preferred_element_type=jnp.float32)
    dn2 = t[:, :16]
    t16 = t[:, 16:32]
    q = jnp.where(t16 >= 0.0, 0.0, t16 / dn2)
    r = h1 - d * jnp.dot(q, st_ref[...], preferred_element_type=jnp.float32)
    h = jnp.dot(r, kn2_ref[...], preferred_element_type=jnp.float32)
    hinv = jnp.sqrt(jnp.dot(h * h, ss_ref[:48, :16],
                            preferred_element_type=jnp.float32) + 1e-12)
    hid = jax.nn.relu(jnp.dot(hinv, wo1_ref[...],
                              preferred_element_type=jnp.float32)
                      + bo1_ref[...])
    o_ref[...] = (jnp.dot(hid, wo2_ref[...], preferred_element_type=jnp.float32)
                  + bo2_ref[...])


def _kron3(w):
    return jnp.kron(w.T, jnp.eye(3, dtype=w.dtype))


def kernel(x, edge_index, edge_attr, node_weight, W1, Wd, W2, Wn1, Wnd, Wn2,
           Wo1, bo1, Wo2, bo2):
    n, nd, _ = x.shape
    e = edge_index.shape[1]
    hid = W2.shape[0]
    src = edge_index[0]
    dst = edge_index[1]

    xt = jnp.concatenate(
        [x.reshape(n, 3 * nd), node_weight[:, None],
         jnp.zeros((n, GW - 3 * nd - 1), jnp.float32)], axis=1)
    src2d = src.reshape(e // IW, IW)
    dst2d = dst.reshape(e // IW, IW)
    ea = edge_attr.reshape(e, -1)
    zeros_blk = jnp.zeros(((-(-n // NS) + 7) // 8 * 8, 32), jnp.float32)

    s_mat = jnp.kron(jnp.eye(hid, dtype=jnp.float32),
                     jnp.ones((3, 1), jnp.float32))
    ss = jnp.zeros((96, 32), jnp.float32)
    ss = ss.at[:48, :16].set(s_mat).at[48:, 16:].set(s_mat)
    st = s_mat.T

    k1 = _kron3(W1)
    kd = _kron3(Wd)
    k2 = _kron3(W2)
    kn1 = _kron3(Wn1)
    knd = _kron3(Wnd)
    kn2 = _kron3(Wn2)

    g = _make_gather(n, e)(xt, src2d)

    be = 4000
    msgs = pl.pallas_call(
        _edge_body,
        grid=(e // be,),
        in_specs=[
            pl.BlockSpec((be, GW), lambda i: (i, 0)),
            pl.BlockSpec((be, 12), lambda i: (i, 0)),
            pl.BlockSpec((60, 48), lambda i: (0, 0)),
            pl.BlockSpec((48, 48), lambda i: (0, 0)),
            pl.BlockSpec((48, 48), lambda i: (0, 0)),
            pl.BlockSpec((96, 32), lambda i: (0, 0)),
            pl.BlockSpec((16, 48), lambda i: (0, 0)),
        ],
        out_specs=pl.BlockSpec((2, be, 32), lambda i: (0, i, 0)),
        out_shape=jax.ShapeDtypeStruct((2, e, 32), jnp.float32),
    )(g, ea, k1, kd, k2, ss, st)

    msums = _make_scatter(n, e)(msgs, dst2d, zeros_blk)

    bn = 2000
    out = pl.pallas_call(
        _node_body,
        grid=(n // bn,),
        in_specs=[
            pl.BlockSpec((bn, 48), lambda i: (i, 0)),
            pl.BlockSpec((2, bn, 32), lambda i: (0, i, 0)),
            pl.BlockSpec((96, 48), lambda i: (0, 0)),
            pl.BlockSpec((48, 48), lambda i: (0, 0)),
            pl.BlockSpec((48, 48), lambda i: (0, 0)),
            pl.BlockSpec((96, 32), lambda i: (0, 0)),
            pl.BlockSpec((16, 48), lambda i: (0, 0)),
            pl.BlockSpec((16, 16), lambda i: (0, 0)),
            pl.BlockSpec((16,), lambda i: (0,)),
            pl.BlockSpec((16, 1), lambda i: (0, 0)),
            pl.BlockSpec((1,), lambda i: (0,)),
        ],
        out_specs=pl.BlockSpec((bn, 1), lambda i: (i, 0)),
        out_shape=jax.ShapeDtypeStruct((n, 1), jnp.float32),
    )(x.reshape(n, 3 * nd), msums, kn1, knd, kn2, ss, st,
      Wo1.T, bo1, Wo2.T, bo2)

    return out


# speedup vs baseline: 47.8406x; 1.2877x over previous
import functools

import jax
import jax.numpy as jnp
from jax import lax
from jax.experimental import pallas as pl
from jax.experimental.pallas import tpu as pltpu
from jax.experimental.pallas import tpu_sc as plsc

NC = 2
NS = 16
GW = 64
IW = 125


def _make_gather(n_nodes, n_edges):
    ew = n_edges // (NC * NS)
    ca = 1000
    sub = ca // IW
    mesh = plsc.VectorSubcoreMesh(core_axis_name="c", subcore_axis_name="s")

    @functools.partial(
        pl.kernel, mesh=mesh,
        out_type=jax.ShapeDtypeStruct((n_edges, GW), jnp.float32),
        scratch_types=[
            pltpu.VMEM((sub, IW), jnp.int32),
            pltpu.VMEM((ca, GW), jnp.float32),
            pltpu.SemaphoreType.DMA,
        ],
        compiler_params=pltpu.CompilerParams(use_tc_tiling_on_sc=False),
    )
    def gather_k(xt_hbm, src2d_hbm, out_hbm, idx_v, rows_v, sem):
        wid = lax.axis_index("s") * NC + lax.axis_index("c")
        base = wid * ew

        def chunk(i, carry):
            off = pl.multiple_of(base + i * ca, 8)
            pltpu.sync_copy(src2d_hbm.at[pl.ds(pl.multiple_of(off // IW, 8), sub)],
                            idx_v)
            cps = [
                pltpu.async_copy(
                    xt_hbm.at[idx_v.at[j]],
                    rows_v.at[pl.ds(j * IW, IW)], sem)
                for j in range(sub)
            ]
            for cp in cps:
                cp.wait()
            pltpu.sync_copy(rows_v, out_hbm.at[pl.ds(off, ca)])
            return carry

        lax.fori_loop(0, ew // ca, chunk, 0)

    return gather_k


def _make_scatter(n_nodes, n_edges):
    es = n_edges // NS
    cc = 500
    sub = cc // IW
    nra = (-(-n_nodes // NS) + 7) // 8 * 8
    nrl = n_nodes - (NS - 1) * nra
    mesh = plsc.VectorSubcoreMesh(core_axis_name="c", subcore_axis_name="s")

    @functools.partial(
        pl.kernel, mesh=mesh,
        out_type=jax.ShapeDtypeStruct((n_nodes, 2 * 32), jnp.float32),
        scratch_types=[
            pltpu.VMEM_SHARED((n_nodes, 32), jnp.float32),
            pltpu.VMEM((sub, IW), jnp.int32),
            pltpu.VMEM((cc, 32), jnp.float32),
            pltpu.SemaphoreType.DMA,
        ],
        compiler_params=pltpu.CompilerParams(use_tc_tiling_on_sc=False),
    )
    def scatter_k(msgs_hbm, dst2d_hbm, zeros_hbm, out_hbm, acc, idx_v, buf, sem):
        c = lax.axis_index("c")
        s = lax.axis_index("s")

        @pl.when(s < NS - 1)
        def _():
            pltpu.sync_copy(zeros_hbm,
                            acc.at[pl.ds(pl.multiple_of(s * nra, 8), nra)])

        @pl.when(s == NS - 1)
        def _():
            pltpu.sync_copy(zeros_hbm.at[pl.ds(0, nrl)],
                            acc.at[pl.ds((NS - 1) * nra, nrl)])

        plsc.subcore_barrier()

        base = s * es
        col = pl.multiple_of(c * 32, 8)

        def chunk(i, carry):
            off = pl.multiple_of(base + i * cc, 8)
            pltpu.sync_copy(dst2d_hbm.at[pl.ds(pl.multiple_of(off // IW, 8), sub)],
                            idx_v)
            pltpu.sync_copy(msgs_hbm.at[pl.ds(off, cc), pl.ds(col, 32)], buf)
            cps = [
                pltpu.async_copy(
                    buf.at[pl.ds(j * IW, IW)],
                    acc.at[idx_v.at[j]], sem, add=True)
                for j in range(sub)
            ]
            for cp in cps:
                cp.wait()
            return carry

        lax.fori_loop(0, es // cc, chunk, 0)
        plsc.subcore_barrier()

        @pl.when(s < NS - 1)
        def _():
            off = pl.multiple_of(s * nra, 8)
            pltpu.sync_copy(acc.at[pl.ds(off, nra)],
                            out_hbm.at[pl.ds(off, nra), pl.ds(col, 32)])

        @pl.when(s == NS - 1)
        def _():
            pltpu.sync_copy(acc.at[pl.ds((NS - 1) * nra, nrl)],
                            out_hbm.at[pl.ds((NS - 1) * nra, nrl), pl.ds(col, 32)])

    return scatter_k


def _edge_body(g_ref, ea_ref, a1_ref, a1d_ref, gg_ref, cpad_ref, cmask_ref,
               cvec_ref, a2_ref, o_ref):
    f32 = jnp.float32
    b = g_ref.shape[0]
    z48 = jnp.zeros((b, 48), f32)
    z4 = jnp.zeros((b, 4), f32)
    g = g_ref[...] + jnp.concatenate([z48, ea_ref[...], z4], axis=1)
    h1 = jnp.dot(g, a1_ref[...], preferred_element_type=f32)
    hd = jnp.dot(g, a1d_ref[...], preferred_element_type=f32)
    u = jnp.dot(hd * hd, gg_ref[...], preferred_element_type=f32) + cpad_ref[...]
    v = jnp.dot(h1 * hd, gg_ref[...], preferred_element_type=f32)
    qb = jnp.minimum(v, 0.0) / u
    r = h1 - hd * qb
    m = jnp.dot(r, a2_ref[...], preferred_element_type=f32)
    wvec = g[:, 63:64] * cmask_ref[...] + cvec_ref[...]
    o_ref[...] = m * wvec


def _node_body(x_ref, nm_ref, bw_ref, cden_ref, an1x_ref, an1a_ref, and_ref,
               gg_ref, cpad_ref, an2_ref, gs_ref, wo1_ref, bo1_ref, wo2_ref,
               bo2_ref, o_ref):
    f32 = jnp.float32
    nm = nm_ref[...]
    u2 = jnp.dot(nm, bw_ref[...], preferred_element_type=f32) + cden_ref[...]
    agg = nm / u2
    h1 = (jnp.dot(x_ref[...], an1x_ref[...], preferred_element_type=f32)
          + jnp.dot(agg, an1a_ref[...], preferred_element_type=f32))
    hd = jnp.dot(h1, and_ref[...], preferred_element_type=f32)
    u = jnp.dot(hd * hd, gg_ref[...], preferred_element_type=f32) + cpad_ref[...]
    v = jnp.dot(h1 * hd, gg_ref[...], preferred_element_type=f32)
    qb = jnp.minimum(v, 0.0) / u
    r = h1 - hd * qb
    h = jnp.dot(r, an2_ref[...], preferred_element_type=f32)
    hinv = jnp.sqrt(jnp.dot(h * h, gs_ref[...], preferred_element_type=f32)
                    + 1e-12)
    hid = jax.nn.relu(jnp.dot(hinv, wo1_ref[...], preferred_element_type=f32)
                      + bo1_ref[...])
    o_ref[...] = (jnp.dot(hid, wo2_ref[...], preferred_element_type=f32)
                  + bo2_ref[...])


def _kron3(w):
    return jnp.kron(w.T, jnp.eye(3, dtype=w.dtype))


def kernel(x, edge_index, edge_attr, node_weight, W1, Wd, W2, Wn1, Wnd, Wn2,
           Wo1, bo1, Wo2, bo2):
    n, nd, _ = x.shape
    e = edge_index.shape[1]
    hid = W2.shape[0]
    f = 3 * hid
    src = edge_index[0]
    dst = edge_index[1]

    xt = jnp.concatenate(
        [x.reshape(n, 3 * nd), jnp.zeros((n, GW - 3 * nd - 1), jnp.float32),
         node_weight[:, None]], axis=1)
    src2d = src.reshape(e // IW, IW)
    dst2d = dst.reshape(e // IW, IW)
    ea = edge_attr.reshape(e, -1)
    zeros_blk = jnp.zeros(((-(-n // NS) + 7) // 8 * 8, 32), jnp.float32)

    k1 = _kron3(W1)
    a1 = jnp.zeros((GW, GW), jnp.float32).at[:f, :f].set(k1[:f])
    a1 = a1.at[f:f + 12, :f].set(k1[f:])
    a1 = a1.at[63, f].set(1.0)
    ad = jnp.zeros((GW, GW), jnp.float32).at[:f, :f].set(_kron3(Wd))
    ad = ad.at[f, f].set(1.0)
    a1d = a1 @ ad
    gg = jnp.zeros((GW, GW), jnp.float32).at[:f, :f].set(
        jnp.kron(jnp.eye(hid, dtype=jnp.float32), jnp.ones((3, 3), jnp.float32)))
    cpad = jnp.zeros((1, GW), jnp.float32).at[0, f:].set(1.0)
    cmask = jnp.zeros((1, GW), jnp.float32).at[0, :f].set(1.0)
    cvec = jnp.zeros((1, GW), jnp.float32).at[0, f].set(1.0)
    a2 = jnp.zeros((GW, GW), jnp.float32).at[:f, :f].set(_kron3(W2))
    a2 = a2.at[f, f].set(1.0)
    bw = jnp.zeros((GW, GW), jnp.float32).at[f, :f].set(1.0)

    kn1 = _kron3(Wn1)
    an1x = jnp.zeros((f, GW), jnp.float32).at[:, :f].set(kn1[:f])
    an1a = jnp.zeros((GW, GW), jnp.float32).at[:f, :f].set(kn1[f:])
    and_ = jnp.zeros((GW, GW), jnp.float32).at[:f, :f].set(_kron3(Wnd))
    an2 = jnp.zeros((GW, GW), jnp.float32).at[:f, :f].set(_kron3(Wn2))
    cden = jnp.zeros((1, GW), jnp.float32).at[0, :f].set(1e-12).at[0, f:].set(1.0)
    gs = jnp.zeros((GW, hid), jnp.float32).at[:f, :].set(
        jnp.kron(jnp.eye(hid, dtype=jnp.float32), jnp.ones((3, 1), jnp.float32)))

    g = _make_gather(n, e)(xt, src2d)

    be = 8000
    full = lambda i: (0, 0)
    msgs = pl.pallas_call(
        _edge_body,
        grid=(e // be,),
        in_specs=[
            pl.BlockSpec((be, GW), lambda i: (i, 0)),
            pl.BlockSpec((be, 12), lambda i: (i, 0)),
            pl.BlockSpec((GW, GW), full),
            pl.BlockSpec((GW, GW), full),
            pl.BlockSpec((GW, GW), full),
            pl.BlockSpec((1, GW), full),
            pl.BlockSpec((1, GW), full),
            pl.BlockSpec((1, GW), full),
            pl.BlockSpec((GW, GW), full),
        ],
        out_specs=pl.BlockSpec((be, GW), lambda i: (i, 0)),
        out_shape=jax.ShapeDtypeStruct((e, GW), jnp.float32),
    )(g, ea, a1, a1d, gg, cpad, cmask, cvec, a2)

    nm = _make_scatter(n, e)(msgs, dst2d, zeros_blk)

    bn = 5000
    out = pl.pallas_call(
        _node_body,
        grid=(n // bn,),
        in_specs=[
            pl.BlockSpec((bn, f), lambda i: (i, 0)),
            pl.BlockSpec((bn, GW), lambda i: (i, 0)),
            pl.BlockSpec((GW, GW), full),
            pl.BlockSpec((1, GW), full),
            pl.BlockSpec((f, GW), full),
            pl.BlockSpec((GW, GW), full),
            pl.BlockSpec((GW, GW), full),
            pl.BlockSpec((GW, GW), full),
            pl.BlockSpec((1, GW), full),
            pl.BlockSpec((GW, GW), full),
            pl.BlockSpec((GW, hid), full),
            pl.BlockSpec((hid, hid), full),
            pl.BlockSpec((hid,), lambda i: (0,)),
            pl.BlockSpec((hid, 1), full),
            pl.BlockSpec((1,), lambda i: (0,)),
        ],
        out_specs=pl.BlockSpec((bn, 1), lambda i: (i, 0)),
        out_shape=jax.ShapeDtypeStruct((n, 1), jnp.float32),
    )(x.reshape(n, f), nm, bw, cden, an1x, an1a, and_, gg, cpad, an2, gs,
      Wo1.T, bo1, Wo2.T, bo2)

    return out
